# baseline (speedup 1.0000x reference)
import functools

import jax
import jax.numpy as jnp
from jax import lax
from jax.experimental import pallas as pl
from jax.experimental.pallas import tpu as pltpu
from jax.experimental.pallas import tpu_sc as plsc

_NC = 2
_NS = 16
_NW = _NC * _NS
_CH = 32


def _make_sc_gather(V, H, N, S, O):
    per_w = N // _NW
    n_chunks = per_w // _CH
    assert n_chunks % 2 == 0
    mesh = plsc.VectorSubcoreMesh(core_axis_name="c", subcore_axis_name="s")

    @functools.partial(
        pl.kernel,
        out_type=jax.ShapeDtypeStruct((N, H), jnp.float32),
        mesh=mesh,
        scratch_types=[
            pltpu.VMEM((per_w,), jnp.int32),
            pltpu.VMEM((_CH,), jnp.int32),
            pltpu.VMEM((_CH,), jnp.int32),
            pltpu.VMEM((_CH,), jnp.int32),
            pltpu.VMEM((_CH,), jnp.int32),
            pltpu.VMEM((_CH,), jnp.int32),
            pltpu.VMEM((_CH,), jnp.int32),
            pltpu.VMEM((_CH,), jnp.int32),
            pltpu.VMEM((_CH,), jnp.int32),
            pltpu.VMEM((_CH, H), jnp.float32),
            pltpu.VMEM((_CH, H), jnp.float32),
            pltpu.SemaphoreType.DMA,
            pltpu.SemaphoreType.DMA,
        ],
    )
    def sc_gather(cv_hbm, ocr_hbm, idx_hbm, out_hbm,
                  idx_v, cvi0, cvi1, bi0, bi1, oi0, oi1, t0, t1,
                  cvb0, cvb1, gsem0, gsem1):
        wid = lax.axis_index("s") * _NC + lax.axis_index("c")
        base = wid * per_w
        pltpu.sync_copy(idx_hbm.at[pl.ds(base, per_w)], idx_v)
        iota16 = lax.iota(jnp.int32, 16)
        s_vec = jnp.full((16,), S, jnp.int32)
        slots = ((cvi0, bi0, oi0, t0, cvb0, gsem0),
                 (cvi1, bi1, oi1, t1, cvb1, gsem1))

        def prep(k, slot):
            cvi, bi, oi, tt, cvb, gsem = slot
            row0 = k * _CH
            for j in range(_CH // 16):
                off = row0 + 16 * j
                i = idx_v[pl.ds(off, 16)]
                m = i >= V
                r = base + off + iota16
                b = lax.div(r, s_vec)
                cvi[pl.ds(16 * j, 16)] = jnp.where(m, 0, i)
                bi[pl.ds(16 * j, 16)] = b
                oi[pl.ds(16 * j, 16)] = jnp.where(m, i - V, 0)
                tt[pl.ds(16 * j, 16)] = jnp.where(m, 1, 0)
            pltpu.make_async_copy(cv_hbm.at[cvi], cvb, gsem).start()

        def fin(k, slot):
            cvi, bi, oi, tt, cvb, gsem = slot
            row0 = k * _CH
            m_acc = jnp.zeros((16,), jnp.int32)
            for j in range(_CH // 16):
                m_acc = m_acc | tt[pl.ds(16 * j, 16)]
            any_s = m_acc[0]
            for l in range(1, 16):
                any_s = any_s | m_acc[l]
            pltpu.make_async_copy(cv_hbm.at[cvi], cvb, gsem).wait()

            @pl.when(any_s > 0)
            def _ocr_merge():
                def rbody(gg, rcarry):
                    tv = tt[pl.ds(gg * 16, 16)]
                    bv = bi[pl.ds(gg * 16, 16)]
                    ov = oi[pl.ds(gg * 16, 16)]
                    for j in range(16):
                        rr = gg * 16 + j

                        @pl.when(tv[j] != 0)
                        def _fetch_row(rr=rr, j=j):
                            pltpu.sync_copy(ocr_hbm.at[bv[j], ov[j]],
                                            cvb.at[rr])

                    return rcarry

                lax.fori_loop(0, _CH // 16, rbody, 0)

            pltpu.sync_copy(cvb, out_hbm.at[pl.ds(base + row0, _CH)])

        prep(0, slots[0])

        def cbody(t, carry):
            k0 = 2 * t
            prep(k0 + 1, slots[1])
            fin(k0, slots[0])

            @pl.when(t < n_chunks // 2 - 1)
            def _more():
                prep(k0 + 2, slots[0])

            fin(k0 + 1, slots[1])
            return carry

        lax.fori_loop(0, n_chunks // 2, cbody, 0)

    return sc_gather


def _tc_ln(raw_p, prev_inds, pos50, type0, g_cv, b_cv, g_ocr, b_ocr, g_e, b_e, V):
    B, S, _ = prev_inds.shape
    H = raw_p.shape[-1]
    BB = 8

    def kfn(x_ref, ind_ref, pos_ref, ty_ref, gcv_ref, bcv_ref,
            gocr_ref, bocr_ref, ge_ref, be_ref, o_ref):
        x = x_ref[...].reshape(BB, S, H)
        mu = jnp.mean(x, -1, keepdims=True)
        var = jnp.mean(jnp.square(x - mu), -1, keepdims=True)
        xn = (x - mu) * lax.rsqrt(var + 1e-5)
        m = ind_ref[...] >= V
        g = jnp.where(m, gocr_ref[...][None], gcv_ref[...][None])
        bta = jnp.where(m, bocr_ref[...][None], bcv_ref[...][None])
        y = xn * g + bta

        pt = pos_ref[...] + ty_ref[...]
        pmu = jnp.mean(pt, -1, keepdims=True)
        pvar = jnp.mean(jnp.square(pt - pmu), -1, keepdims=True)
        ptn = (pt - pmu) * lax.rsqrt(pvar + 1e-5) * ge_ref[...] + be_ref[...]
        o_ref[...] = y + ptn[None]

    return pl.pallas_call(
        kfn,
        grid=(B // BB,),
        in_specs=[
            pl.BlockSpec((BB * S, H), lambda i: (i, 0)),
            pl.BlockSpec((BB, S, 1), lambda i: (i, 0, 0)),
            pl.BlockSpec((S, H), lambda i: (0, 0)),
            pl.BlockSpec((1, H), lambda i: (0, 0)),
            pl.BlockSpec((1, H), lambda i: (0, 0)),
            pl.BlockSpec((1, H), lambda i: (0, 0)),
            pl.BlockSpec((1, H), lambda i: (0, 0)),
            pl.BlockSpec((1, H), lambda i: (0, 0)),
            pl.BlockSpec((1, H), lambda i: (0, 0)),
            pl.BlockSpec((1, H), lambda i: (0, 0)),
        ],
        out_specs=pl.BlockSpec((BB, S, H), lambda i: (i, 0, 0)),
        out_shape=jax.ShapeDtypeStruct((B, S, H), jnp.float32),
    )(raw_p, prev_inds, pos50, type0, g_cv, b_cv, g_ocr, b_ocr, g_e, b_e)


def kernel(common_voc_embedding, ocr_embedding, prev_inds, pos_emb, type_emb,
           ln_cv_g, ln_cv_b, ln_ocr_g, ln_ocr_b, ln_emb_g, ln_emb_b):
    V, H = common_voc_embedding.shape
    B, S = prev_inds.shape
    O = ocr_embedding.shape[1]
    N = B * S

    idx = prev_inds.reshape(N).astype(jnp.int32)


    raw_p = _make_sc_gather(V, H, N, S, O)(common_voc_embedding, ocr_embedding, idx)

    r2 = lambda v: v.reshape(1, H)
    return _tc_ln(raw_p, prev_inds.reshape(B, S, 1).astype(jnp.int32),
                  pos_emb[:S], type_emb[0:1], r2(ln_cv_g), r2(ln_cv_b),
                  r2(ln_ocr_g), r2(ln_ocr_b), r2(ln_emb_g), r2(ln_emb_b), V)


# speedup vs baseline: 1.2017x; 1.2017x over previous
import functools

import jax
import jax.numpy as jnp
from jax import lax
from jax.experimental import pallas as pl
from jax.experimental.pallas import tpu as pltpu
from jax.experimental.pallas import tpu_sc as plsc

_NC = 2
_NS = 16
_NW = _NC * _NS
_CH = 32


def _make_sc_gather(V, H, N, S, O):
    per_w = N // _NW
    n_chunks = per_w // _CH
    assert n_chunks % 2 == 0
    mesh = plsc.VectorSubcoreMesh(core_axis_name="c", subcore_axis_name="s")

    @functools.partial(
        pl.kernel,
        out_type=jax.ShapeDtypeStruct((N, H), jnp.float32),
        mesh=mesh,
        scratch_types=[
            pltpu.VMEM((per_w,), jnp.int32),
            pltpu.VMEM((_CH,), jnp.int32),
            pltpu.VMEM((_CH,), jnp.int32),
            pltpu.VMEM((_CH,), jnp.int32),
            pltpu.VMEM((_CH,), jnp.int32),
            pltpu.VMEM((_CH,), jnp.int32),
            pltpu.VMEM((_CH,), jnp.int32),
            pltpu.VMEM((_CH,), jnp.int32),
            pltpu.VMEM((_CH,), jnp.int32),
            pltpu.VMEM((_CH, H), jnp.float32),
            pltpu.VMEM((_CH, H), jnp.float32),
            pltpu.SemaphoreType.DMA,
            pltpu.SemaphoreType.DMA,
        ],
    )
    def sc_gather(cv_hbm, ocr_hbm, idx_hbm, out_hbm,
                  idx_v, cvi0, cvi1, bi0, bi1, oi0, oi1, t0, t1,
                  cvb0, cvb1, gsem0, gsem1):
        wid = lax.axis_index("s") * _NC + lax.axis_index("c")
        base = wid * per_w
        pltpu.sync_copy(idx_hbm.at[pl.ds(base, per_w)], idx_v)
        iota16 = lax.iota(jnp.int32, 16)
        s_vec = jnp.full((16,), S, jnp.int32)
        slots = ((cvi0, bi0, oi0, t0, cvb0, gsem0),
                 (cvi1, bi1, oi1, t1, cvb1, gsem1))

        def prep(k, slot):
            cvi, bi, oi, tt, cvb, gsem = slot
            row0 = k * _CH
            for j in range(_CH // 16):
                off = row0 + 16 * j
                i = idx_v[pl.ds(off, 16)]
                m = i >= V
                r = base + off + iota16
                b = lax.div(r, s_vec)
                cvi[pl.ds(16 * j, 16)] = jnp.where(m, 0, i)
                bi[pl.ds(16 * j, 16)] = b
                oi[pl.ds(16 * j, 16)] = jnp.where(m, i - V, 0)
                tt[pl.ds(16 * j, 16)] = jnp.where(m, 1, 0)
            pltpu.make_async_copy(cv_hbm.at[cvi], cvb, gsem).start()

        def fin(k, slot):
            cvi, bi, oi, tt, cvb, gsem = slot
            row0 = k * _CH
            m_acc = jnp.zeros((16,), jnp.int32)
            for j in range(_CH // 16):
                m_acc = m_acc | tt[pl.ds(16 * j, 16)]
            any_s = m_acc[0]
            for l in range(1, 16):
                any_s = any_s | m_acc[l]
            pltpu.make_async_copy(cv_hbm.at[cvi], cvb, gsem).wait()

            @pl.when(any_s > 0)
            def _ocr_merge():
                def rbody(gg, rcarry):
                    tv = tt[pl.ds(gg * 16, 16)]
                    bv = bi[pl.ds(gg * 16, 16)]
                    ov = oi[pl.ds(gg * 16, 16)]
                    for j in range(16):
                        rr = gg * 16 + j

                        @pl.when(tv[j] != 0)
                        def _fetch_row(rr=rr, j=j):
                            pltpu.sync_copy(ocr_hbm.at[bv[j], ov[j]],
                                            cvb.at[rr])

                    return rcarry

                lax.fori_loop(0, _CH // 16, rbody, 0)

            pltpu.sync_copy(cvb, out_hbm.at[pl.ds(base + row0, _CH)])

        prep(0, slots[0])

        def cbody(t, carry):
            k0 = 2 * t
            prep(k0 + 1, slots[1])
            fin(k0, slots[0])

            @pl.when(t < n_chunks // 2 - 1)
            def _more():
                prep(k0 + 2, slots[0])

            fin(k0 + 1, slots[1])
            return carry

        lax.fori_loop(0, n_chunks // 2, cbody, 0)

    return sc_gather


def _tc_ln(raw_p, prev_inds, pos50, type0, g_cv, b_cv, g_ocr, b_ocr, g_e, b_e, V):
    B, S, _ = prev_inds.shape
    H = raw_p.shape[-1]
    BB = 8

    def kfn(x_ref, ind_ref, pos_ref, ty_ref, gcv_ref, bcv_ref,
            gocr_ref, bocr_ref, ge_ref, be_ref, o_ref):
        x = x_ref[...].reshape(BB, S, H)
        mu = jnp.mean(x, -1, keepdims=True)
        var = jnp.mean(jnp.square(x - mu), -1, keepdims=True)
        xn = (x - mu) * lax.rsqrt(var + 1e-5)
        m = ind_ref[...] >= V
        g = jnp.where(m, gocr_ref[...][None], gcv_ref[...][None])
        bta = jnp.where(m, bocr_ref[...][None], bcv_ref[...][None])
        y = xn * g + bta

        pt = pos_ref[...] + ty_ref[...]
        pmu = jnp.mean(pt, -1, keepdims=True)
        pvar = jnp.mean(jnp.square(pt - pmu), -1, keepdims=True)
        ptn = (pt - pmu) * lax.rsqrt(pvar + 1e-5) * ge_ref[...] + be_ref[...]
        o_ref[...] = jnp.transpose(y + ptn[None], (1, 0, 2))

    return pl.pallas_call(
        kfn,
        grid=(B // BB,),
        in_specs=[
            pl.BlockSpec((BB * S, H), lambda i: (i, 0)),
            pl.BlockSpec((BB, S, 1), lambda i: (i, 0, 0)),
            pl.BlockSpec((S, H), lambda i: (0, 0)),
            pl.BlockSpec((1, H), lambda i: (0, 0)),
            pl.BlockSpec((1, H), lambda i: (0, 0)),
            pl.BlockSpec((1, H), lambda i: (0, 0)),
            pl.BlockSpec((1, H), lambda i: (0, 0)),
            pl.BlockSpec((1, H), lambda i: (0, 0)),
            pl.BlockSpec((1, H), lambda i: (0, 0)),
            pl.BlockSpec((1, H), lambda i: (0, 0)),
        ],
        out_specs=pl.BlockSpec((S, BB, H), lambda i: (0, i, 0)),
        out_shape=jax.ShapeDtypeStruct((S, B, H), jnp.float32),
    )(raw_p, prev_inds, pos50, type0, g_cv, b_cv, g_ocr, b_ocr, g_e, b_e)


def kernel(common_voc_embedding, ocr_embedding, prev_inds, pos_emb, type_emb,
           ln_cv_g, ln_cv_b, ln_ocr_g, ln_ocr_b, ln_emb_g, ln_emb_b):
    V, H = common_voc_embedding.shape
    B, S = prev_inds.shape
    O = ocr_embedding.shape[1]
    N = B * S

    idx = prev_inds.reshape(N).astype(jnp.int32)


    raw_p = _make_sc_gather(V, H, N, S, O)(common_voc_embedding, ocr_embedding, idx)

    r2 = lambda v: v.reshape(1, H)
    out_t = _tc_ln(raw_p, prev_inds.reshape(B, S, 1).astype(jnp.int32),
                   pos_emb[:S], type_emb[0:1], r2(ln_cv_g), r2(ln_cv_b),
                   r2(ln_ocr_g), r2(ln_ocr_b), r2(ln_emb_g), r2(ln_emb_b), V)
    return jnp.transpose(out_t, (1, 0, 2))
